# baseline (speedup 1.0000x reference)
import jax
import jax.numpy as jnp
from jax import lax
from jax.experimental import pallas as pl
from jax.experimental.pallas import tpu as pltpu
from jax.experimental.pallas import tpu_sc as plsc

_T = 8
_LAT = 721
_LON = 1440
_LANES = 16
_NSUB = 16
_CHUNK = 48


def _dyn_gather(x, perm):
    dnums = lax.GatherDimensionNumbers(
        offset_dims=(), collapsed_slice_dims=(0,), start_index_map=(0,))
    return lax.gather(x, perm[:, None], dnums, (1,),
                      mode=lax.GatherScatterMode.PROMISE_IN_BOUNDS)


def _coupler_body(tvec_hbm, qvec_hbm, buf_sst, buf_ice, new_sst, new_ice,
                  out, tv_v, qv_v, row_v):
    pltpu.sync_copy(tvec_hbm, tv_v)
    pltpu.sync_copy(qvec_hbm, qv_v)
    diff = jnp.abs(tv_v[...] - qv_v[...])
    lanes = lax.iota(jnp.int32, _LANES)
    vals, args = diff, lanes
    for off in (1, 2, 4, 8):
        perm = lax.bitwise_and(lanes + off, _LANES - 1)
        ov = _dyn_gather(vals, perm)
        oa = _dyn_gather(args, perm)
        take = (ov < vals) | ((ov == vals) & (oa < args))
        vals = jnp.where(take, ov, vals)
        args = jnp.where(take, oa, args)
    idx = args[0]
    is_new = idx == (_T - 1)
    srow = idx + 1

    c = lax.axis_index("c")
    s = lax.axis_index("s")

    def field(buf, new, slot):
        def copy_chunk(r0, nrows, dst):
            @pl.when(is_new)
            def _():
                pltpu.sync_copy(new.at[pl.ds(r0, nrows)], dst)

            @pl.when(jnp.logical_not(is_new))
            def _():
                pltpu.sync_copy(buf.at[srow, pl.ds(r0, nrows)], dst)

            pltpu.sync_copy(dst, out.at[slot, pl.ds(r0, nrows)])

        @pl.when(s < _NSUB - 1)
        def _():
            copy_chunk(pl.multiple_of(s * _CHUNK, 8), _CHUNK, row_v)

        @pl.when(s == _NSUB - 1)
        def _():
            copy_chunk(_LAT - 1, 1, row_v.at[pl.ds(0, 1)])

    @pl.when(c == 0)
    def _():
        field(buf_sst, new_sst, 0)

    @pl.when(c == 1)
    def _():
        field(buf_ice, new_ice, 1)


def kernel(buf_sst, buf_ice, times_buf, new_sst, new_ice, new_time, query_time):
    pad = jnp.full((_LANES - _T,), jnp.inf, dtype=jnp.float32)
    tvec = jnp.concatenate([times_buf[1:], new_time, pad])
    qvec = jnp.broadcast_to(query_time, (_LANES,))

    mesh = plsc.VectorSubcoreMesh(core_axis_name="c", subcore_axis_name="s")
    fn = pl.kernel(
        _coupler_body,
        mesh=mesh,
        out_type=jax.ShapeDtypeStruct((2, _LAT, _LON), jnp.float32),
        scratch_types=[
            pltpu.VMEM((_LANES,), jnp.float32),
            pltpu.VMEM((_LANES,), jnp.float32),
            pltpu.VMEM((_CHUNK, _LON), jnp.float32),
        ],
    )
    return fn(tvec, qvec, buf_sst, buf_ice, new_sst, new_ice)
